# baseline (speedup 1.0000x reference)
import functools

import jax
import jax.numpy as jnp
from jax import lax
from jax.experimental import pallas as pl
from jax.experimental.pallas import tpu as pltpu
from jax.experimental.pallas import tpu_sc as plsc

_NC = 2
_NS = 16
_NW = _NC * _NS
_L = 16

_ROW = 128
_CE = 4096
_RPC = _CE // _ROW


def _softplus_neg_abs(x):
    u = jnp.exp(-jnp.abs(x))
    t = u / (u + 2.0)
    t2 = t * t
    p = 1.0 + t2 * (1.0 / 3.0 + t2 * (0.2 + t2 * (1.0 / 7.0 + t2 * (1.0 / 9.0))))
    return 2.0 * t * p


def _bce_pair(a, b):
    pa = a > 0.0
    pb = b > 0.0
    na = a < 0.0
    nb = b < 0.0
    m = jnp.where((pa != pb) | (na != nb), 1.0, 0.0)
    t0 = jnp.where(pb, 1.0, 0.0)
    t1 = jnp.where(pa, 1.0, 0.0)
    bce = (jnp.maximum(a, 0.0) - a * t0 + _softplus_neg_abs(a)
           + jnp.maximum(b, 0.0) - b * t1 + _softplus_neg_abs(b))
    return bce, m


def _acc_pairs(val, n_edges, carry):
    def pair(i, c):
        al, ac = c
        off = i * _L
        a = val[pl.ds(off, _L)]
        b = val[pl.ds(_CE + off, _L)]
        bce, m = _bce_pair(a, b)
        return (al + bce * m, ac + m)

    return lax.fori_loop(0, n_edges // _L, pair, carry)


def _sc_body(n_edges, sdf_hbm, edges_hbm, out_hbm,
             idx0, idx1, val0, val1, out_v, sem0, sem1):
    wid = lax.axis_index("s") * _NC + lax.axis_index("c")
    total_chunks = n_edges // _CE
    tail_edges = n_edges % _CE
    total_pairs = total_chunks // 2
    odd_chunk = total_chunks % 2
    base_p = total_pairs // _NW
    rem_p = total_pairs % _NW
    my_pairs = base_p + jnp.where(wid < rem_p, 1, 0)
    chunk0 = 2 * (wid * base_p + jnp.minimum(wid, rem_p))

    def issue(cidx, idx_ref, val_ref, sem, n_rows=_RPC):
        pltpu.sync_copy(edges_hbm.at[:, pl.ds(cidx * _CE, n_rows * _ROW)],
                        idx_ref.at[:, pl.ds(0, n_rows * _ROW)])
        for half in (0, 1):
            for r in range(n_rows):
                pltpu.async_copy(
                    sdf_hbm.at[idx_ref.at[half, pl.ds(r * _ROW, _ROW)]],
                    val_ref.at[pl.ds(half * _CE + r * _ROW, _ROW)], sem)

    def drain(idx_ref, val_ref, sem, n_rows=_RPC):
        for half in (0, 1):
            for r in range(n_rows):
                pltpu.make_async_copy(
                    sdf_hbm.at[idx_ref.at[half, pl.ds(r * _ROW, _ROW)]],
                    val_ref.at[pl.ds(half * _CE + r * _ROW, _ROW)],
                    sem).wait()

    @pl.when(my_pairs > 0)
    def _():
        issue(chunk0, idx0, val0, sem0)

    def pair(p, acc):
        issue(chunk0 + 2 * p + 1, idx1, val1, sem1)
        drain(idx0, val0, sem0)
        acc = _acc_pairs(val0, _CE, acc)

        @pl.when(p + 1 < my_pairs)
        def _():
            issue(chunk0 + 2 * p + 2, idx0, val0, sem0)

        drain(idx1, val1, sem1)
        return _acc_pairs(val1, _CE, acc)

    zeros = jnp.zeros((_L,), jnp.float32)
    acc = lax.fori_loop(0, my_pairs, pair, (zeros, zeros))

    if odd_chunk:
        def odd(_, carry):
            issue(total_chunks - 1, idx0, val0, sem0)
            drain(idx0, val0, sem0)
            return _acc_pairs(val0, _CE, carry)
        acc = lax.fori_loop(0, jnp.where(wid == _NW - 1, 1, 0), odd, acc)

    if tail_edges:
        tail_rows = tail_edges // _ROW

        def tail(_, carry):
            e0 = total_chunks * _CE
            pltpu.sync_copy(edges_hbm.at[:, pl.ds(e0, tail_edges)],
                            idx0.at[:, pl.ds(0, tail_edges)])
            for half in (0, 1):
                for r in range(tail_rows):
                    pltpu.async_copy(
                        sdf_hbm.at[idx0.at[half, pl.ds(r * _ROW, _ROW)]],
                        val0.at[pl.ds(half * _CE + r * _ROW, _ROW)], sem0)
            drain(idx0, val0, sem0, n_rows=tail_rows)
            return _acc_pairs(val0, tail_edges, carry)

        acc = lax.fori_loop(0, jnp.where(wid == 0, 1, 0), tail, acc)

    acc_l, acc_c = acc
    out_v[0, :] = acc_l
    out_v[1, :] = acc_c
    pltpu.sync_copy(out_v, out_hbm.at[wid])


def kernel(sdf, all_edges):
    e = all_edges.shape[0]
    assert e % _ROW == 0 and all_edges.shape[1] == 2
    edges = all_edges.T

    mesh = plsc.VectorSubcoreMesh(core_axis_name="c", subcore_axis_name="s")
    run = pl.kernel(
        functools.partial(_sc_body, e),
        out_type=jax.ShapeDtypeStruct((_NW, 2, _L), jnp.float32),
        mesh=mesh,
        scratch_types=[
            pltpu.VMEM((2, _CE), jnp.int32),
            pltpu.VMEM((2, _CE), jnp.int32),
            pltpu.VMEM((2 * _CE,), jnp.float32),
            pltpu.VMEM((2 * _CE,), jnp.float32),
            pltpu.VMEM((2, _L), jnp.float32),
            pltpu.SemaphoreType.DMA,
            pltpu.SemaphoreType.DMA,
        ],
        compiler_params=pltpu.CompilerParams(needs_layout_passes=False),
    )
    parts = run(sdf, edges)
    loss = jnp.sum(parts[:, 0, :])
    cnt = jnp.sum(parts[:, 1, :])
    return loss / jnp.maximum(cnt, 1.0)


# speedup vs baseline: 1.0144x; 1.0144x over previous
import functools

import jax
import jax.numpy as jnp
from jax import lax
from jax.experimental import pallas as pl
from jax.experimental.pallas import tpu as pltpu
from jax.experimental.pallas import tpu_sc as plsc

_NC = 2
_NS = 16
_NW = _NC * _NS
_L = 16

_ROW = 128
_CE = 2048
_RPC = _CE // _ROW


def _softplus_neg_abs(x):
    u = jnp.exp(-jnp.abs(x))
    t = u / (u + 2.0)
    t2 = t * t
    p = 1.0 + t2 * (1.0 / 3.0 + t2 * (0.2 + t2 * (1.0 / 7.0 + t2 * (1.0 / 9.0))))
    return 2.0 * t * p


def _bce_pair(a, b):
    pa = a > 0.0
    pb = b > 0.0
    na = a < 0.0
    nb = b < 0.0
    m = jnp.where((pa != pb) | (na != nb), 1.0, 0.0)
    t0 = jnp.where(pb, 1.0, 0.0)
    t1 = jnp.where(pa, 1.0, 0.0)
    bce = (jnp.maximum(a, 0.0) - a * t0 + _softplus_neg_abs(a)
           + jnp.maximum(b, 0.0) - b * t1 + _softplus_neg_abs(b))
    return bce, m


def _acc_pairs(val, n_edges, carry):
    def pair(i, c):
        al, ac = c
        off = i * _L
        a = val[pl.ds(off, _L)]
        b = val[pl.ds(_CE + off, _L)]
        bce, m = _bce_pair(a, b)
        return (al + bce * m, ac + m)

    return lax.fori_loop(0, n_edges // _L, pair, carry)


def _sc_body(n_edges, sdf_hbm, edges_hbm, out_hbm,
             idx0, idx1, val0, val1, out_v, sem0, sem1):
    wid = lax.axis_index("s") * _NC + lax.axis_index("c")
    total_chunks = n_edges // _CE
    tail_edges = n_edges % _CE
    total_pairs = total_chunks // 2
    odd_chunk = total_chunks % 2
    base_p = total_pairs // _NW
    rem_p = total_pairs % _NW
    my_pairs = base_p + jnp.where(wid < rem_p, 1, 0)
    chunk0 = 2 * (wid * base_p + jnp.minimum(wid, rem_p))

    def issue(cidx, idx_ref, val_ref, sem, n_rows=_RPC):
        pltpu.sync_copy(edges_hbm.at[:, pl.ds(cidx * _CE, n_rows * _ROW)],
                        idx_ref.at[:, pl.ds(0, n_rows * _ROW)])
        for half in (0, 1):
            for r in range(n_rows):
                pltpu.async_copy(
                    sdf_hbm.at[idx_ref.at[half, pl.ds(r * _ROW, _ROW)]],
                    val_ref.at[pl.ds(half * _CE + r * _ROW, _ROW)], sem)

    def drain(idx_ref, val_ref, sem, n_rows=_RPC):
        for half in (0, 1):
            for r in range(n_rows):
                pltpu.make_async_copy(
                    sdf_hbm.at[idx_ref.at[half, pl.ds(r * _ROW, _ROW)]],
                    val_ref.at[pl.ds(half * _CE + r * _ROW, _ROW)],
                    sem).wait()

    @pl.when(my_pairs > 0)
    def _():
        issue(chunk0, idx0, val0, sem0)

    def pair(p, acc):
        issue(chunk0 + 2 * p + 1, idx1, val1, sem1)
        drain(idx0, val0, sem0)
        acc = _acc_pairs(val0, _CE, acc)

        @pl.when(p + 1 < my_pairs)
        def _():
            issue(chunk0 + 2 * p + 2, idx0, val0, sem0)

        drain(idx1, val1, sem1)
        return _acc_pairs(val1, _CE, acc)

    zeros = jnp.zeros((_L,), jnp.float32)
    acc = lax.fori_loop(0, my_pairs, pair, (zeros, zeros))

    if odd_chunk:
        def odd(_, carry):
            issue(total_chunks - 1, idx0, val0, sem0)
            drain(idx0, val0, sem0)
            return _acc_pairs(val0, _CE, carry)
        acc = lax.fori_loop(0, jnp.where(wid == _NW - 1, 1, 0), odd, acc)

    if tail_edges:
        tail_rows = tail_edges // _ROW

        def tail(_, carry):
            e0 = total_chunks * _CE
            pltpu.sync_copy(edges_hbm.at[:, pl.ds(e0, tail_edges)],
                            idx0.at[:, pl.ds(0, tail_edges)])
            for half in (0, 1):
                for r in range(tail_rows):
                    pltpu.async_copy(
                        sdf_hbm.at[idx0.at[half, pl.ds(r * _ROW, _ROW)]],
                        val0.at[pl.ds(half * _CE + r * _ROW, _ROW)], sem0)
            drain(idx0, val0, sem0, n_rows=tail_rows)
            return _acc_pairs(val0, tail_edges, carry)

        acc = lax.fori_loop(0, jnp.where(wid == 0, 1, 0), tail, acc)

    acc_l, acc_c = acc
    out_v[0, :] = acc_l
    out_v[1, :] = acc_c
    pltpu.sync_copy(out_v, out_hbm.at[wid])


def kernel(sdf, all_edges):
    e = all_edges.shape[0]
    assert e % _ROW == 0 and all_edges.shape[1] == 2
    edges = all_edges.T

    mesh = plsc.VectorSubcoreMesh(core_axis_name="c", subcore_axis_name="s")
    run = pl.kernel(
        functools.partial(_sc_body, e),
        out_type=jax.ShapeDtypeStruct((_NW, 2, _L), jnp.float32),
        mesh=mesh,
        scratch_types=[
            pltpu.VMEM((2, _CE), jnp.int32),
            pltpu.VMEM((2, _CE), jnp.int32),
            pltpu.VMEM((2 * _CE,), jnp.float32),
            pltpu.VMEM((2 * _CE,), jnp.float32),
            pltpu.VMEM((2, _L), jnp.float32),
            pltpu.SemaphoreType.DMA,
            pltpu.SemaphoreType.DMA,
        ],
        compiler_params=pltpu.CompilerParams(needs_layout_passes=False),
    )
    parts = run(sdf, edges)
    loss = jnp.sum(parts[:, 0, :])
    cnt = jnp.sum(parts[:, 1, :])
    return loss / jnp.maximum(cnt, 1.0)
